# baseline (speedup 1.0000x reference)
import functools

import jax
import jax.numpy as jnp
from jax import lax
from jax.experimental import pallas as pl
from jax.experimental.pallas import tpu as pltpu
from jax.experimental.pallas import tpu_sc as plsc

NUM_FIELDS = 26
VOCAB = 100000
EMBED_DIM = 32
BATCH = 16384

NC = 2
NS = 16
NW = NC * NS
LANES = 16

PAIRS = NUM_FIELDS * EMBED_DIM
PER_W = PAIRS // NW
QTR = BATCH // 4
NQ = 4


def _body(xt_hbm, tab_hbm, out_hbm, row_v, idx_v, ob0_v, ob1_v, sem0, sem1, rsem, isem):
    wid = lax.axis_index("s") * NC + lax.axis_index("c")
    obufs = (ob0_v, ob1_v)
    sems = (sem0, sem1)

    def drain(b, p):
        pltpu.make_async_copy(
            obufs[b], out_hbm.at[p, pl.ds(0, QTR)], sems[b]
        ).wait()

    def pair_body(k, carry):
        i = k
        e = wid
        p = i * EMBED_DIM + e

        row_copy = pltpu.async_copy(tab_hbm.at[i, e, :], row_v, rsem)
        idx_copy = pltpu.async_copy(xt_hbm.at[i, :], idx_v, isem)

        @pl.when(k > 0)
        def _():
            drain(0, p)
            drain(1, p)

        idx_copy.wait()
        row_copy.wait()

        for q in range(NQ):
            b = q % 2
            if q >= 2:
                drain(b, p)

            ob = obufs[b]
            qbase = q * QTR

            @plsc.parallel_loop(0, QTR, step=LANES, unroll=16)
            def _(g):
                iv = idx_v[pl.ds(qbase + g, LANES)]
                ob[pl.ds(g, LANES)] = plsc.load_gather(row_v, [iv])

            pltpu.async_copy(ob, out_hbm.at[p, pl.ds(qbase, QTR)], sems[b])
        return carry

    lax.fori_loop(0, PER_W, pair_body, 0)
    drain(0, wid * PER_W)
    drain(1, wid * PER_W)


_mesh = plsc.VectorSubcoreMesh(core_axis_name="c", subcore_axis_name="s")

_gather = functools.partial(
    pl.kernel,
    mesh=_mesh,
    out_type=jax.ShapeDtypeStruct((PAIRS, BATCH), jnp.float32),
    compiler_params=pltpu.CompilerParams(
        use_tc_tiling_on_sc=True, needs_layout_passes=False
    ),
    scratch_types=[
        pltpu.VMEM((VOCAB,), jnp.float32),
        pltpu.VMEM((BATCH,), jnp.int32),
        pltpu.VMEM((QTR,), jnp.float32),
        pltpu.VMEM((QTR,), jnp.float32),
        pltpu.SemaphoreType.DMA,
        pltpu.SemaphoreType.DMA,
        pltpu.SemaphoreType.DMA,
        pltpu.SemaphoreType.DMA,
    ],
)(_body)


@jax.jit
def kernel(x, tables):
    xt = x.T
    tab_t = jnp.transpose(tables, (0, 2, 1))
    out_t = _gather(xt, tab_t)
    return out_t.T


# speedup vs baseline: 1.2406x; 1.2406x over previous
import functools

import jax
import jax.numpy as jnp
from jax import lax
from jax.experimental import pallas as pl
from jax.experimental.pallas import tpu as pltpu
from jax.experimental.pallas import tpu_sc as plsc

NUM_FIELDS = 26
VOCAB = 100000
EMBED_DIM = 32
BATCH = 16384

NC = 2
NS = 16
NW = NC * NS
LANES = 16

PAIRS = NUM_FIELDS * EMBED_DIM
PER_W = PAIRS // NW
QTR = BATCH // 4
NQ = 4


def _body(xt_hbm, tab_hbm, out_hbm, row_v, idx_v, ob0_v, ob1_v, sem0, sem1, rsem):
    wid = lax.axis_index("s") * NC + lax.axis_index("c")
    obufs = (ob0_v, ob1_v)
    sems = (sem0, sem1)

    def drain(b, p):
        pltpu.make_async_copy(
            obufs[b], out_hbm.at[p, pl.ds(0, QTR)], sems[b]
        ).wait()

    def pair_body(k, carry):
        p = wid * PER_W + k
        i = p // EMBED_DIM
        e = p % EMBED_DIM

        @pl.when(jnp.logical_or(k == 0, i != (p - 1) // EMBED_DIM))
        def _():
            pltpu.sync_copy(xt_hbm.at[i, :], idx_v)

        row_copy = pltpu.async_copy(tab_hbm.at[i, e, :], row_v, rsem)

        @pl.when(k > 0)
        def _():
            drain(0, p)
            drain(1, p)

        row_copy.wait()

        for q in range(NQ):
            b = q % 2
            if q >= 2:
                drain(b, p)

            ob = obufs[b]
            qbase = q * QTR

            @plsc.parallel_loop(0, QTR, step=LANES, unroll=16)
            def _(g):
                iv = idx_v[pl.ds(qbase + g, LANES)]
                ob[pl.ds(g, LANES)] = plsc.load_gather(row_v, [iv])

            pltpu.async_copy(ob, out_hbm.at[p, pl.ds(qbase, QTR)], sems[b])
        return carry

    lax.fori_loop(0, PER_W, pair_body, 0)
    drain(0, wid * PER_W)
    drain(1, wid * PER_W)


_mesh = plsc.VectorSubcoreMesh(core_axis_name="c", subcore_axis_name="s")

_gather = functools.partial(
    pl.kernel,
    mesh=_mesh,
    out_type=jax.ShapeDtypeStruct((PAIRS, BATCH), jnp.float32),
    compiler_params=pltpu.CompilerParams(
        use_tc_tiling_on_sc=True, needs_layout_passes=False
    ),
    scratch_types=[
        pltpu.VMEM((VOCAB,), jnp.float32),
        pltpu.VMEM((BATCH,), jnp.int32),
        pltpu.VMEM((QTR,), jnp.float32),
        pltpu.VMEM((QTR,), jnp.float32),
        pltpu.SemaphoreType.DMA,
        pltpu.SemaphoreType.DMA,
        pltpu.SemaphoreType.DMA,
    ],
)(_body)


@jax.jit
def kernel(x, tables):
    xt = x.T
    tab_t = jnp.transpose(tables, (0, 2, 1))
    out_t = _gather(xt, tab_t)
    return out_t.T
